# baseline (speedup 1.0000x reference)
import functools

import jax
import jax.numpy as jnp
from jax import lax
from jax.experimental import pallas as pl
from jax.experimental.pallas import tpu as pltpu
from jax.experimental.pallas import tpu_sc as plsc

_NUM_TOKENS = 16384
_D_MODEL = 2048
_NUM_EXPERTS = 16
_BLOCK_T = 1024
_GRID = _NUM_TOKENS // _BLOCK_T

_NUM_WORKERS = 32
_TOK_PER_W = _NUM_TOKENS // _NUM_WORKERS
_LANES = 16


def _tc_body(x_ref, w_ref, lt_ref, loss_ref, acc_ref):
    step = pl.program_id(0)

    @pl.when(step == 0)
    def _init():
        acc_ref[...] = jnp.zeros_like(acc_ref)

    w = w_ref[...]
    logits = jax.lax.dot_general(
        x_ref[...], w, (((1,), (1,)), ((), ())),
        preferred_element_type=jnp.float32)
    lt = logits.T
    lt_ref[:, pl.ds(step * _BLOCK_T, _BLOCK_T)] = lt

    m = jnp.max(lt, axis=0, keepdims=True)
    e = jnp.exp(lt - m)
    s = jnp.sum(e, axis=0, keepdims=True)
    acc_ref[...] += jnp.sum(e / s, axis=1, keepdims=True)

    @pl.when(step == _GRID - 1)
    def _fin():
        p = acc_ref[...] / _NUM_TOKENS
        loss_ref[0, 0] = jnp.sum(p * jnp.log(p + 1e-8))


def _tc_logits(x, W):
    return pl.pallas_call(
        _tc_body,
        grid=(_GRID,),
        in_specs=[
            pl.BlockSpec((_BLOCK_T, _D_MODEL), lambda i: (i, 0)),
            pl.BlockSpec((_NUM_EXPERTS, _D_MODEL), lambda i: (0, 0)),
        ],
        out_specs=[
            pl.BlockSpec((_NUM_EXPERTS, _NUM_TOKENS), lambda i: (0, 0)),
            pl.BlockSpec(memory_space=pltpu.SMEM, block_shape=(1, 1),
                         index_map=lambda i: (0, 0)),
        ],
        out_shape=[
            jax.ShapeDtypeStruct((_NUM_EXPERTS, _NUM_TOKENS), jnp.float32),
            jax.ShapeDtypeStruct((1, 1), jnp.float32),
        ],
        scratch_shapes=[pltpu.VMEM((_NUM_EXPERTS, 1), jnp.float32)],
        compiler_params=pltpu.CompilerParams(
            dimension_semantics=("arbitrary",)),
    )(x, W)


@functools.partial(
    pl.kernel,
    out_type=[
        jax.ShapeDtypeStruct((2, _NUM_TOKENS), jnp.float32),
        jax.ShapeDtypeStruct((2, _NUM_TOKENS), jnp.int32),
    ],
    mesh=plsc.VectorSubcoreMesh(core_axis_name="c", subcore_axis_name="s"),
    scratch_types=[
        pltpu.VMEM((_NUM_EXPERTS, _TOK_PER_W), jnp.float32),
        pltpu.VMEM((2, _TOK_PER_W), jnp.float32),
        pltpu.VMEM((2, _TOK_PER_W), jnp.int32),
    ],
)
def _sc_route(lt_hbm, ts_hbm, ti_hbm, lt_v, ts_v, ti_v):
    wid = lax.axis_index("s") * 2 + lax.axis_index("c")
    base = wid * _TOK_PER_W
    pltpu.sync_copy(lt_hbm.at[:, pl.ds(base, _TOK_PER_W)], lt_v)

    def group(g, carry):
        col = g * _LANES
        rows = [lt_v[e, pl.ds(col, _LANES)] for e in range(_NUM_EXPERTS)]
        m = rows[0]
        for e in range(1, _NUM_EXPERTS):
            m = jnp.maximum(m, rows[e])
        exps = [jnp.exp(r - m) for r in rows]
        v1 = exps[0]
        for e in range(1, _NUM_EXPERTS):
            v1 = jnp.maximum(v1, exps[e])
        i1 = jnp.full((_LANES,), 16.0, dtype=jnp.float32)
        for e in range(_NUM_EXPERTS - 1, -1, -1):
            i1 = jnp.where(exps[e] == v1, jnp.float32(e), i1)
        neg = jnp.full((_LANES,), -jnp.inf, dtype=jnp.float32)
        v2 = neg
        for e in range(_NUM_EXPERTS):
            v2 = jnp.maximum(v2, jnp.where(i1 == jnp.float32(e), neg, exps[e]))
        i2 = jnp.full((_LANES,), 16.0, dtype=jnp.float32)
        for e in range(_NUM_EXPERTS - 1, -1, -1):
            hit = jnp.logical_and(exps[e] == v2, i1 != jnp.float32(e))
            i2 = jnp.where(hit, jnp.float32(e), i2)
        den = v1 + v2
        ts_v[0, pl.ds(col, _LANES)] = v1 / den
        ts_v[1, pl.ds(col, _LANES)] = v2 / den
        ti_v[0, pl.ds(col, _LANES)] = i1.astype(jnp.int32)
        ti_v[1, pl.ds(col, _LANES)] = i2.astype(jnp.int32)
        return carry

    lax.fori_loop(0, _TOK_PER_W // _LANES, group, 0)
    pltpu.sync_copy(ts_v, ts_hbm.at[:, pl.ds(base, _TOK_PER_W)])
    pltpu.sync_copy(ti_v, ti_hbm.at[:, pl.ds(base, _TOK_PER_W)])


def kernel(x, W):
    lt, loss = _tc_logits(x, W)
    ts_t, ti_t = _sc_route(lt)
    return ts_t.T, ti_t.T, loss.reshape(())


# speedup vs baseline: 1.4530x; 1.4530x over previous
import jax
import jax.numpy as jnp
from jax.experimental import pallas as pl
from jax.experimental.pallas import tpu as pltpu

_NUM_TOKENS = 16384
_D_MODEL = 2048
_NUM_EXPERTS = 16
_BLOCK_T = 1024
_GRID = _NUM_TOKENS // _BLOCK_T


def _moe_gate_body(x_ref, w_ref, ts_ref, ti_ref, loss_ref, acc_ref):
    step = pl.program_id(0)

    @pl.when(step == 0)
    def _init():
        acc_ref[...] = jnp.zeros_like(acc_ref)

    w = w_ref[...]
    logits = jax.lax.dot_general(
        x_ref[...], w, (((1,), (1,)), ((), ())),
        preferred_element_type=jnp.float32)
    lt = logits.T

    m = jnp.max(lt, axis=0, keepdims=True)
    e = jnp.exp(lt - m)
    s = jnp.sum(e, axis=0, keepdims=True)
    scores = e / s

    acc_ref[...] += jnp.sum(scores, axis=1, keepdims=True)

    row = jax.lax.broadcasted_iota(jnp.int32, scores.shape, 0)
    v1 = jnp.max(scores, axis=0, keepdims=True)
    i1 = jnp.min(jnp.where(scores == v1, row, _NUM_EXPERTS),
                 axis=0, keepdims=True)
    masked = jnp.where(row == i1, -jnp.inf, scores)
    v2 = jnp.max(masked, axis=0, keepdims=True)
    i2 = jnp.min(jnp.where(masked == v2, row, _NUM_EXPERTS),
                 axis=0, keepdims=True)

    denom = v1 + v2
    ts_ref[...] = jnp.concatenate([v1 / denom, v2 / denom], axis=0)
    ti_ref[...] = jnp.concatenate([i1, i2], axis=0)

    @pl.when(step == _GRID - 1)
    def _fin():
        p = acc_ref[...] / _NUM_TOKENS
        loss_ref[0, 0] = jnp.sum(p * jnp.log(p + 1e-8))


def kernel(x, W):
    ts_t, ti_t, loss = pl.pallas_call(
        _moe_gate_body,
        grid=(_GRID,),
        in_specs=[
            pl.BlockSpec((_BLOCK_T, _D_MODEL), lambda i: (i, 0)),
            pl.BlockSpec((_NUM_EXPERTS, _D_MODEL), lambda i: (0, 0)),
        ],
        out_specs=[
            pl.BlockSpec((2, _BLOCK_T), lambda i: (0, i)),
            pl.BlockSpec((2, _BLOCK_T), lambda i: (0, i)),
            pl.BlockSpec(memory_space=pltpu.SMEM, block_shape=(1, 1),
                         index_map=lambda i: (0, 0)),
        ],
        out_shape=[
            jax.ShapeDtypeStruct((2, _NUM_TOKENS), jnp.float32),
            jax.ShapeDtypeStruct((2, _NUM_TOKENS), jnp.int32),
            jax.ShapeDtypeStruct((1, 1), jnp.float32),
        ],
        scratch_shapes=[pltpu.VMEM((_NUM_EXPERTS, 1), jnp.float32)],
        compiler_params=pltpu.CompilerParams(
            dimension_semantics=("arbitrary",)),
    )(x, W)
    return ts_t.T, ti_t.T, loss.reshape(())
